# baseline (speedup 1.0000x reference)
import functools

import jax
import jax.numpy as jnp
from jax import lax
from jax.experimental import pallas as pl
from jax.experimental.pallas import tpu as pltpu
from jax.experimental.pallas import tpu_sc as plsc

B = 4
N = 4096
M = 4096
NBLK = 8
RPB = N // NBLK
KV = M // 16
UQ = 4
UJ = 8
BIG = 3.0e38


def _sc_chamfer(qm2, qraw, km, kraw):
    mesh = plsc.VectorSubcoreMesh(core_axis_name="c", subcore_axis_name="s")

    @functools.partial(
        pl.kernel,
        mesh=mesh,
        out_type=[
            jax.ShapeDtypeStruct((B, N, 16), jnp.float32),
            jax.ShapeDtypeStruct((B, NBLK, M), jnp.float32),
        ],
        scratch_types=[
            pltpu.VMEM((3, RPB), jnp.float32),
            pltpu.VMEM((3, RPB), jnp.float32),
            pltpu.VMEM((3, M), jnp.float32),
            pltpu.VMEM((3, M), jnp.float32),
            pltpu.VMEM((RPB,), jnp.float32),
            pltpu.VMEM((M,), jnp.float32),
            pltpu.VMEM((M,), jnp.float32),
            pltpu.VMEM((RPB, 16), jnp.float32),
        ],
    )
    def sck(qm2_h, qraw_h, km_h, kraw_h, d1m_h, d2p_h,
            qm2_v, qraw_v, km_v, kraw_v, qs_v, s2_v, d2_v, d1_v):
        c = lax.axis_index("c")
        s = lax.axis_index("s")
        w = s * 2 + c
        b = w // NBLK
        r = w % NBLK
        base = r * RPB

        pltpu.sync_copy(qm2_h.at[b, :, pl.ds(base, RPB)], qm2_v)
        pltpu.sync_copy(qraw_h.at[b, :, pl.ds(base, RPB)], qraw_v)
        pltpu.sync_copy(km_h.at[b], km_v)
        pltpu.sync_copy(kraw_h.at[b], kraw_v)

        def qs_body(i, _):
            x = qraw_v[0, pl.ds(i * 16, 16)]
            y = qraw_v[1, pl.ds(i * 16, 16)]
            z = qraw_v[2, pl.ds(i * 16, 16)]
            qs_v[pl.ds(i * 16, 16)] = x * x + y * y + z * z
            return 0

        lax.fori_loop(0, RPB // 16, qs_body, 0)

        def s2_body(j, _):
            x = kraw_v[0, pl.ds(j * 16, 16)]
            y = kraw_v[1, pl.ds(j * 16, 16)]
            z = kraw_v[2, pl.ds(j * 16, 16)]
            s2_v[pl.ds(j * 16, 16)] = x * x + y * y + z * z
            d2_v[pl.ds(j * 16, 16)] = jnp.full((16,), BIG, jnp.float32)
            return 0

        lax.fori_loop(0, KV, s2_body, 0)

        def qc_body(qc, _):
            q0 = qc * 16
            qs_c = qs_v[pl.ds(q0, 16)]
            qx_c = qm2_v[0, pl.ds(q0, 16)]
            qy_c = qm2_v[1, pl.ds(q0, 16)]
            qz_c = qm2_v[2, pl.ds(q0, 16)]
            for g in range(16 // UQ):
                qs_u = [qs_c[g * UQ + u] for u in range(UQ)]
                qx_u = [qx_c[g * UQ + u] for u in range(UQ)]
                qy_u = [qy_c[g * UQ + u] for u in range(UQ)]
                qz_u = [qz_c[g * UQ + u] for u in range(UQ)]

                def j_body(jc, m1s):
                    outs = list(m1s)
                    for jj in range(UJ):
                        j = jc * UJ + jj
                        kx = km_v[0, pl.ds(j * 16, 16)]
                        ky = km_v[1, pl.ds(j * 16, 16)]
                        kz = km_v[2, pl.ds(j * 16, 16)]
                        s2 = s2_v[pl.ds(j * 16, 16)]
                        vs = []
                        for u in range(UQ):
                            v = ((s2 + qs_u[u]) + qx_u[u] * kx) + \
                                (qy_u[u] * ky + qz_u[u] * kz)
                            vs.append(v)
                            outs[u] = jnp.minimum(outs[u], v)
                        wmin = jnp.minimum(jnp.minimum(vs[0], vs[1]),
                                           jnp.minimum(vs[2], vs[3]))
                        d2_v[pl.ds(j * 16, 16)] = jnp.minimum(
                            d2_v[pl.ds(j * 16, 16)], wmin)
                    return tuple(outs)

                init = tuple(jnp.full((16,), BIG, jnp.float32)
                             for _ in range(UQ))
                m1f = lax.fori_loop(0, KV // UJ, j_body, init)
                for u in range(UQ):
                    qi = q0 + g * UQ + u
                    d1_v[qi, :] = m1f[u]
            return 0

        lax.fori_loop(0, RPB // 16, qc_body, 0)

        pltpu.sync_copy(d1_v, d1m_h.at[b, pl.ds(base, RPB)])
        pltpu.sync_copy(d2_v, d2p_h.at[b, r])

    return sck(qm2, qraw, km, kraw)


def _merge_body(d1m_ref, d2p_ref, out_ref):
    d1 = jnp.maximum(jnp.min(d1m_ref[...], axis=2), 0.0)
    s1 = jnp.sum(jnp.sqrt(d1))
    d2 = jnp.maximum(jnp.min(d2p_ref[...], axis=1), 0.0)
    s2 = jnp.sum(jnp.sqrt(d2))
    out_ref[0, 0] = s1 * (0.5 / (B * N)) + s2 * (0.5 / (B * M))


def _merge(d1m, d2p):
    out = pl.pallas_call(
        _merge_body,
        out_specs=pl.BlockSpec(memory_space=pltpu.SMEM),
        out_shape=jax.ShapeDtypeStruct((1, 1), jnp.float32),
    )(d1m, d2p)
    return out[0, 0]


@jax.jit
def kernel(pcs1, pcs2):
    p1t = jnp.transpose(pcs1, (0, 2, 1))
    p2t = jnp.transpose(pcs2, (0, 2, 1))
    qm2 = lax.reduce_precision(p1t, 8, 7) * -2.0
    km = lax.reduce_precision(p2t, 8, 7)
    d1m, d2p = _sc_chamfer(qm2, p1t, km, p2t)
    return _merge(d1m, d2p)


# speedup vs baseline: 1.0569x; 1.0569x over previous
import functools

import jax
import jax.numpy as jnp
from jax import lax
from jax.experimental import pallas as pl
from jax.experimental.pallas import tpu as pltpu
from jax.experimental.pallas import tpu_sc as plsc

B = 4
N = 4096
M = 4096
NBLK = 8
RPB = N // NBLK
KV = M // 16
UQ = 4
UJ = 8
BIG = 3.0e38


def _sc_chamfer(qm2, qraw, km, kraw):
    mesh = plsc.VectorSubcoreMesh(core_axis_name="c", subcore_axis_name="s")

    @functools.partial(
        pl.kernel,
        mesh=mesh,
        out_type=[
            jax.ShapeDtypeStruct((B, N, 16), jnp.float32),
            jax.ShapeDtypeStruct((B, NBLK, M), jnp.float32),
        ],
        scratch_types=[
            pltpu.VMEM((3, RPB), jnp.float32),
            pltpu.VMEM((3, RPB), jnp.float32),
            pltpu.VMEM((3, M), jnp.float32),
            pltpu.VMEM((3, M), jnp.float32),
            pltpu.VMEM((RPB,), jnp.float32),
            pltpu.VMEM((M,), jnp.float32),
            pltpu.VMEM((M,), jnp.float32),
            pltpu.VMEM((RPB, 16), jnp.float32),
        ],
    )
    def sck(qm2_h, qraw_h, km_h, kraw_h, d1m_h, d2p_h,
            qm2_v, qraw_v, km_v, kraw_v, qs_v, s2_v, d2_v, d1_v):
        c = lax.axis_index("c")
        s = lax.axis_index("s")
        w = s * 2 + c
        b = w // NBLK
        r = w % NBLK
        base = r * RPB

        pltpu.sync_copy(qm2_h.at[b, :, pl.ds(base, RPB)], qm2_v)
        pltpu.sync_copy(qraw_h.at[b, :, pl.ds(base, RPB)], qraw_v)
        pltpu.sync_copy(km_h.at[b], km_v)
        pltpu.sync_copy(kraw_h.at[b], kraw_v)

        def qs_body(i, _):
            x = qraw_v[0, pl.ds(i * 16, 16)]
            y = qraw_v[1, pl.ds(i * 16, 16)]
            z = qraw_v[2, pl.ds(i * 16, 16)]
            qs_v[pl.ds(i * 16, 16)] = x * x + y * y + z * z
            return 0

        lax.fori_loop(0, RPB // 16, qs_body, 0)

        def s2_body(j, _):
            x = kraw_v[0, pl.ds(j * 16, 16)]
            y = kraw_v[1, pl.ds(j * 16, 16)]
            z = kraw_v[2, pl.ds(j * 16, 16)]
            s2_v[pl.ds(j * 16, 16)] = x * x + y * y + z * z
            d2_v[pl.ds(j * 16, 16)] = jnp.full((16,), BIG, jnp.float32)
            return 0

        lax.fori_loop(0, KV, s2_body, 0)

        def qc_body(qc, _):
            q0 = qc * 16
            qs_c = qs_v[pl.ds(q0, 16)]
            qx_c = qm2_v[0, pl.ds(q0, 16)]
            qy_c = qm2_v[1, pl.ds(q0, 16)]
            qz_c = qm2_v[2, pl.ds(q0, 16)]
            for g in range(16 // UQ):
                qs_u = [qs_c[g * UQ + u] for u in range(UQ)]
                qx_u = [qx_c[g * UQ + u] for u in range(UQ)]
                qy_u = [qy_c[g * UQ + u] for u in range(UQ)]
                qz_u = [qz_c[g * UQ + u] for u in range(UQ)]

                def j_body(j, m1s):
                    kx = km_v[0, pl.ds(j * 16, 16)]
                    ky = km_v[1, pl.ds(j * 16, 16)]
                    kz = km_v[2, pl.ds(j * 16, 16)]
                    s2 = s2_v[pl.ds(j * 16, 16)]
                    vs = []
                    outs = []
                    for u in range(UQ):
                        v = ((s2 + qs_u[u]) + qx_u[u] * kx) + \
                            (qy_u[u] * ky + qz_u[u] * kz)
                        vs.append(v)
                        outs.append(jnp.minimum(m1s[u], v))
                    wmin = jnp.minimum(jnp.minimum(vs[0], vs[1]),
                                       jnp.minimum(vs[2], vs[3]))
                    d2_v[pl.ds(j * 16, 16)] = jnp.minimum(
                        d2_v[pl.ds(j * 16, 16)], wmin)
                    return tuple(outs)

                init = tuple(jnp.full((16,), BIG, jnp.float32)
                             for _ in range(UQ))
                m1f = plsc.parallel_loop(0, KV, carry=init, unroll=UJ)(j_body)
                for u in range(UQ):
                    qi = q0 + g * UQ + u
                    d1_v[qi, :] = m1f[u]
            return 0

        lax.fori_loop(0, RPB // 16, qc_body, 0)

        pltpu.sync_copy(d1_v, d1m_h.at[b, pl.ds(base, RPB)])
        pltpu.sync_copy(d2_v, d2p_h.at[b, r])

    return sck(qm2, qraw, km, kraw)


def _merge_body(d1m_ref, d2p_ref, out_ref):
    d1 = jnp.maximum(jnp.min(d1m_ref[...], axis=2), 0.0)
    s1 = jnp.sum(jnp.sqrt(d1))
    d2 = jnp.maximum(jnp.min(d2p_ref[...], axis=1), 0.0)
    s2 = jnp.sum(jnp.sqrt(d2))
    out_ref[0, 0] = s1 * (0.5 / (B * N)) + s2 * (0.5 / (B * M))


def _merge(d1m, d2p):
    out = pl.pallas_call(
        _merge_body,
        out_specs=pl.BlockSpec(memory_space=pltpu.SMEM),
        out_shape=jax.ShapeDtypeStruct((1, 1), jnp.float32),
    )(d1m, d2p)
    return out[0, 0]


@jax.jit
def kernel(pcs1, pcs2):
    p1t = jnp.transpose(pcs1, (0, 2, 1))
    p2t = jnp.transpose(pcs2, (0, 2, 1))
    qm2 = lax.reduce_precision(p1t, 8, 7) * -2.0
    km = lax.reduce_precision(p2t, 8, 7)
    d1m, d2p = _sc_chamfer(qm2, p1t, km, p2t)
    return _merge(d1m, d2p)


# speedup vs baseline: 5.6718x; 5.3664x over previous
import functools

import jax
import jax.numpy as jnp
from jax import lax
from jax.experimental import pallas as pl
from jax.experimental.pallas import tpu as pltpu
from jax.experimental.pallas import tpu_sc as plsc

B = 4
N = 4096
M = 4096
NSC = 512
NTC = N - NSC
TN = NTC // 2
NBLK = 8
RPB = NSC // NBLK
KV = M // 16
UQ = 4
UJ = 8
BIG = 3.0e38


def _tc_body(p1_ref, p1s_ref, p2_ref, s1_ref, cmin_ref):
    b = pl.program_id(0)
    i = pl.program_id(1)

    am2 = p1_ref[0]
    a = p1s_ref[0]
    k = p2_ref[0]

    sq1 = jnp.sum(a * a, axis=0)
    sq2 = jnp.sum(k * k, axis=0)
    innerm2 = jax.lax.dot_general(
        am2.astype(jnp.bfloat16), k.astype(jnp.bfloat16),
        (((0,), (0,)), ((), ())),
        preferred_element_type=jnp.float32,
        precision=jax.lax.Precision.DEFAULT,
    )
    d = (sq1[:, None] + sq2[None, :]) + innerm2

    @pl.when(jnp.logical_and(b == 0, i == 0))
    def _():
        s1_ref[0, 0] = 0.0

    d1 = jnp.maximum(jnp.min(d, axis=1), 0.0)
    s1_ref[0, 0] += jnp.sum(jnp.sqrt(d1))

    colmin = jnp.min(d, axis=0)

    @pl.when(i == 0)
    def _():
        cmin_ref[0, 0, :] = colmin

    @pl.when(i > 0)
    def _():
        cmin_ref[0, 0, :] = jnp.minimum(cmin_ref[0, 0, :], colmin)


def _tc_part(p1m2, p1t, p2t):
    return pl.pallas_call(
        _tc_body,
        grid=(B, NTC // TN),
        in_specs=[
            pl.BlockSpec((1, 3, TN), lambda b, i: (b, 0, i)),
            pl.BlockSpec((1, 3, TN), lambda b, i: (b, 0, i)),
            pl.BlockSpec((1, 3, M), lambda b, i: (b, 0, 0)),
        ],
        out_specs=[
            pl.BlockSpec((1, 1), lambda b, i: (0, 0),
                         memory_space=pltpu.SMEM),
            pl.BlockSpec((1, 1, M), lambda b, i: (b, 0, 0)),
        ],
        out_shape=[
            jax.ShapeDtypeStruct((1, 1), jnp.float32),
            jax.ShapeDtypeStruct((B, 1, M), jnp.float32),
        ],
    )(p1m2, p1t, p2t)


def _sc_part(qm2, qraw, km, kraw):
    mesh = plsc.VectorSubcoreMesh(core_axis_name="c", subcore_axis_name="s")

    @functools.partial(
        pl.kernel,
        mesh=mesh,
        out_type=[
            jax.ShapeDtypeStruct((B, NSC, 16), jnp.float32),
            jax.ShapeDtypeStruct((B, NBLK, M), jnp.float32),
        ],
        scratch_types=[
            pltpu.VMEM((3, RPB), jnp.float32),
            pltpu.VMEM((3, RPB), jnp.float32),
            pltpu.VMEM((3, M), jnp.float32),
            pltpu.VMEM((3, M), jnp.float32),
            pltpu.VMEM((RPB,), jnp.float32),
            pltpu.VMEM((M,), jnp.float32),
            pltpu.VMEM((M,), jnp.float32),
            pltpu.VMEM((RPB, 16), jnp.float32),
        ],
    )
    def sck(qm2_h, qraw_h, km_h, kraw_h, d1m_h, d2p_h,
            qm2_v, qraw_v, km_v, kraw_v, qs_v, s2_v, d2_v, d1_v):
        c = lax.axis_index("c")
        s = lax.axis_index("s")
        w = s * 2 + c
        b = w // NBLK
        r = w % NBLK

        pltpu.sync_copy(qm2_h.at[b, r], qm2_v)
        pltpu.sync_copy(qraw_h.at[b, r], qraw_v)
        pltpu.sync_copy(km_h.at[b], km_v)
        pltpu.sync_copy(kraw_h.at[b], kraw_v)

        def qs_body(i, _):
            x = qraw_v[0, pl.ds(i * 16, 16)]
            y = qraw_v[1, pl.ds(i * 16, 16)]
            z = qraw_v[2, pl.ds(i * 16, 16)]
            qs_v[pl.ds(i * 16, 16)] = x * x + y * y + z * z
            return 0

        lax.fori_loop(0, RPB // 16, qs_body, 0)

        def s2_body(j, _):
            x = kraw_v[0, pl.ds(j * 16, 16)]
            y = kraw_v[1, pl.ds(j * 16, 16)]
            z = kraw_v[2, pl.ds(j * 16, 16)]
            s2_v[pl.ds(j * 16, 16)] = x * x + y * y + z * z
            d2_v[pl.ds(j * 16, 16)] = jnp.full((16,), BIG, jnp.float32)
            return 0

        lax.fori_loop(0, KV, s2_body, 0)

        def qc_body(qc, _):
            q0 = qc * 16
            qs_c = qs_v[pl.ds(q0, 16)]
            qx_c = qm2_v[0, pl.ds(q0, 16)]
            qy_c = qm2_v[1, pl.ds(q0, 16)]
            qz_c = qm2_v[2, pl.ds(q0, 16)]
            for g in range(16 // UQ):
                qs_u = [qs_c[g * UQ + u] for u in range(UQ)]
                qx_u = [qx_c[g * UQ + u] for u in range(UQ)]
                qy_u = [qy_c[g * UQ + u] for u in range(UQ)]
                qz_u = [qz_c[g * UQ + u] for u in range(UQ)]

                def j_body(j, m1s):
                    kx = km_v[0, pl.ds(j * 16, 16)]
                    ky = km_v[1, pl.ds(j * 16, 16)]
                    kz = km_v[2, pl.ds(j * 16, 16)]
                    s2 = s2_v[pl.ds(j * 16, 16)]
                    vs = []
                    outs = []
                    for u in range(UQ):
                        v = ((s2 + qs_u[u]) + qx_u[u] * kx) + \
                            (qy_u[u] * ky + qz_u[u] * kz)
                        vs.append(v)
                        outs.append(jnp.minimum(m1s[u], v))
                    wmin = jnp.minimum(jnp.minimum(vs[0], vs[1]),
                                       jnp.minimum(vs[2], vs[3]))
                    d2_v[pl.ds(j * 16, 16)] = jnp.minimum(
                        d2_v[pl.ds(j * 16, 16)], wmin)
                    return tuple(outs)

                init = tuple(jnp.full((16,), BIG, jnp.float32)
                             for _ in range(UQ))
                m1f = plsc.parallel_loop(0, KV, carry=init,
                                         unroll=UJ)(j_body)
                for u in range(UQ):
                    qi = q0 + g * UQ + u
                    d1_v[qi, :] = m1f[u]
            return 0

        lax.fori_loop(0, RPB // 16, qc_body, 0)

        pltpu.sync_copy(d1_v, d1m_h.at[b, pl.ds(r * RPB, RPB)])
        pltpu.sync_copy(d2_v, d2p_h.at[b, r])

    return sck(qm2, qraw, km, kraw)


def _merge_body(s1_ref, cmin_ref, d1m_ref, d2p_ref, out_ref):
    d1 = jnp.maximum(jnp.min(d1m_ref[...], axis=2), 0.0)
    s1 = jnp.sum(s1_ref[...]) + jnp.sum(jnp.sqrt(d1))
    d2 = jnp.minimum(cmin_ref[:, 0, :],
                     jnp.min(d2p_ref[...], axis=1))
    d2 = jnp.maximum(d2, 0.0)
    s2 = jnp.sum(jnp.sqrt(d2))
    out_ref[0, 0] = s1 * (0.5 / (B * N)) + s2 * (0.5 / (B * M))


def _merge(s1, cmin, d1m, d2p):
    out = pl.pallas_call(
        _merge_body,
        out_specs=pl.BlockSpec(memory_space=pltpu.SMEM),
        out_shape=jax.ShapeDtypeStruct((1, 1), jnp.float32),
    )(s1, cmin, d1m, d2p)
    return out[0, 0]


@jax.jit
def kernel(pcs1, pcs2):
    p1t = jnp.transpose(pcs1, (0, 2, 1))
    p2t = jnp.transpose(pcs2, (0, 2, 1))
    p1m2 = p1t * -2.0
    qm2 = lax.reduce_precision(p1t, 8, 7) * -2.0
    km = lax.reduce_precision(p2t, 8, 7)

    qm2_sc = qm2[:, :, NTC:].reshape(B, 3, NBLK, RPB).transpose(0, 2, 1, 3)
    qraw_sc = p1t[:, :, NTC:].reshape(B, 3, NBLK, RPB).transpose(0, 2, 1, 3)
    d1m, d2p = _sc_part(qm2_sc, qraw_sc, km, p2t)
    s1, cmin = _tc_part(p1m2[:, :, :NTC], p1t[:, :, :NTC], p2t)
    return _merge(s1, cmin, d1m, d2p)


# speedup vs baseline: 6.7212x; 1.1850x over previous
import functools

import jax
import jax.numpy as jnp
from jax import lax
from jax.experimental import pallas as pl
from jax.experimental.pallas import tpu as pltpu
from jax.experimental.pallas import tpu_sc as plsc

B = 4
N = 4096
M = 4096
NSC = 256
NTC = N - NSC
TN = NTC // 3
NBLK = 8
RPB = NSC // NBLK
KV = M // 16
UQ = 4
UJ = 8
BIG = 3.0e38


def _tc_body(p1_ref, p1s_ref, p2_ref, s1_ref, cmin_ref):
    b = pl.program_id(0)
    i = pl.program_id(1)

    am2 = p1_ref[0]
    a = p1s_ref[0]
    k = p2_ref[0]

    sq1 = jnp.sum(a * a, axis=0)
    sq2 = jnp.sum(k * k, axis=0)
    innerm2 = jax.lax.dot_general(
        am2.astype(jnp.bfloat16), k.astype(jnp.bfloat16),
        (((0,), (0,)), ((), ())),
        preferred_element_type=jnp.float32,
        precision=jax.lax.Precision.DEFAULT,
    )
    d = (sq1[:, None] + sq2[None, :]) + innerm2

    @pl.when(jnp.logical_and(b == 0, i == 0))
    def _():
        s1_ref[0, 0] = 0.0

    d1 = jnp.maximum(jnp.min(d, axis=1), 0.0)
    s1_ref[0, 0] += jnp.sum(jnp.sqrt(d1))

    colmin = jnp.min(d, axis=0)

    @pl.when(i == 0)
    def _():
        cmin_ref[0, 0, :] = colmin

    @pl.when(i > 0)
    def _():
        cmin_ref[0, 0, :] = jnp.minimum(cmin_ref[0, 0, :], colmin)


def _tc_part(p1m2, p1t, p2t):
    return pl.pallas_call(
        _tc_body,
        grid=(B, NTC // TN),
        in_specs=[
            pl.BlockSpec((1, 3, TN), lambda b, i: (b, 0, i)),
            pl.BlockSpec((1, 3, TN), lambda b, i: (b, 0, i)),
            pl.BlockSpec((1, 3, M), lambda b, i: (b, 0, 0)),
        ],
        out_specs=[
            pl.BlockSpec((1, 1), lambda b, i: (0, 0),
                         memory_space=pltpu.SMEM),
            pl.BlockSpec((1, 1, M), lambda b, i: (b, 0, 0)),
        ],
        out_shape=[
            jax.ShapeDtypeStruct((1, 1), jnp.float32),
            jax.ShapeDtypeStruct((B, 1, M), jnp.float32),
        ],
    )(p1m2, p1t, p2t)


def _sc_part(qm2, qraw, km, kraw):
    mesh = plsc.VectorSubcoreMesh(core_axis_name="c", subcore_axis_name="s")

    @functools.partial(
        pl.kernel,
        mesh=mesh,
        out_type=[
            jax.ShapeDtypeStruct((B, NSC, 16), jnp.float32),
            jax.ShapeDtypeStruct((B, NBLK, M), jnp.float32),
        ],
        scratch_types=[
            pltpu.VMEM((3, RPB), jnp.float32),
            pltpu.VMEM((3, RPB), jnp.float32),
            pltpu.VMEM((3, M), jnp.float32),
            pltpu.VMEM((3, M), jnp.float32),
            pltpu.VMEM((RPB,), jnp.float32),
            pltpu.VMEM((M,), jnp.float32),
            pltpu.VMEM((M,), jnp.float32),
            pltpu.VMEM((RPB, 16), jnp.float32),
        ],
    )
    def sck(qm2_h, qraw_h, km_h, kraw_h, d1m_h, d2p_h,
            qm2_v, qraw_v, km_v, kraw_v, qs_v, s2_v, d2_v, d1_v):
        c = lax.axis_index("c")
        s = lax.axis_index("s")
        w = s * 2 + c
        b = w // NBLK
        r = w % NBLK

        pltpu.sync_copy(qm2_h.at[b, r], qm2_v)
        pltpu.sync_copy(qraw_h.at[b, r], qraw_v)
        pltpu.sync_copy(km_h.at[b], km_v)
        pltpu.sync_copy(kraw_h.at[b], kraw_v)

        def qs_body(i, _):
            x = qraw_v[0, pl.ds(i * 16, 16)]
            y = qraw_v[1, pl.ds(i * 16, 16)]
            z = qraw_v[2, pl.ds(i * 16, 16)]
            qs_v[pl.ds(i * 16, 16)] = x * x + y * y + z * z
            return 0

        lax.fori_loop(0, RPB // 16, qs_body, 0)

        def s2_body(j, _):
            x = kraw_v[0, pl.ds(j * 16, 16)]
            y = kraw_v[1, pl.ds(j * 16, 16)]
            z = kraw_v[2, pl.ds(j * 16, 16)]
            s2_v[pl.ds(j * 16, 16)] = x * x + y * y + z * z
            d2_v[pl.ds(j * 16, 16)] = jnp.full((16,), BIG, jnp.float32)
            return 0

        lax.fori_loop(0, KV, s2_body, 0)

        def qc_body(qc, _):
            q0 = qc * 16
            qs_c = qs_v[pl.ds(q0, 16)]
            qx_c = qm2_v[0, pl.ds(q0, 16)]
            qy_c = qm2_v[1, pl.ds(q0, 16)]
            qz_c = qm2_v[2, pl.ds(q0, 16)]
            for g in range(16 // UQ):
                qs_u = [qs_c[g * UQ + u] for u in range(UQ)]
                qx_u = [qx_c[g * UQ + u] for u in range(UQ)]
                qy_u = [qy_c[g * UQ + u] for u in range(UQ)]
                qz_u = [qz_c[g * UQ + u] for u in range(UQ)]

                def j_body(j, m1s):
                    kx = km_v[0, pl.ds(j * 16, 16)]
                    ky = km_v[1, pl.ds(j * 16, 16)]
                    kz = km_v[2, pl.ds(j * 16, 16)]
                    s2 = s2_v[pl.ds(j * 16, 16)]
                    vs = []
                    outs = []
                    for u in range(UQ):
                        v = ((s2 + qs_u[u]) + qx_u[u] * kx) + \
                            (qy_u[u] * ky + qz_u[u] * kz)
                        vs.append(v)
                        outs.append(jnp.minimum(m1s[u], v))
                    wmin = jnp.minimum(jnp.minimum(vs[0], vs[1]),
                                       jnp.minimum(vs[2], vs[3]))
                    d2_v[pl.ds(j * 16, 16)] = jnp.minimum(
                        d2_v[pl.ds(j * 16, 16)], wmin)
                    return tuple(outs)

                init = tuple(jnp.full((16,), BIG, jnp.float32)
                             for _ in range(UQ))
                m1f = plsc.parallel_loop(0, KV, carry=init,
                                         unroll=UJ)(j_body)
                for u in range(UQ):
                    qi = q0 + g * UQ + u
                    d1_v[qi, :] = m1f[u]
            return 0

        lax.fori_loop(0, RPB // 16, qc_body, 0)

        pltpu.sync_copy(d1_v, d1m_h.at[b, pl.ds(r * RPB, RPB)])
        pltpu.sync_copy(d2_v, d2p_h.at[b, r])

    return sck(qm2, qraw, km, kraw)


def _merge_body(s1_ref, cmin_ref, d1m_ref, d2p_ref, out_ref):
    d1 = jnp.maximum(jnp.min(d1m_ref[...], axis=2), 0.0)
    s1 = jnp.sum(s1_ref[...]) + jnp.sum(jnp.sqrt(d1))
    d2 = jnp.minimum(cmin_ref[:, 0, :],
                     jnp.min(d2p_ref[...], axis=1))
    d2 = jnp.maximum(d2, 0.0)
    s2 = jnp.sum(jnp.sqrt(d2))
    out_ref[0, 0] = s1 * (0.5 / (B * N)) + s2 * (0.5 / (B * M))


def _merge(s1, cmin, d1m, d2p):
    out = pl.pallas_call(
        _merge_body,
        out_specs=pl.BlockSpec(memory_space=pltpu.SMEM),
        out_shape=jax.ShapeDtypeStruct((1, 1), jnp.float32),
    )(s1, cmin, d1m, d2p)
    return out[0, 0]


@jax.jit
def kernel(pcs1, pcs2):
    p1t = jnp.transpose(pcs1, (0, 2, 1))
    p2t = jnp.transpose(pcs2, (0, 2, 1))
    p1m2 = p1t * -2.0
    qm2 = lax.reduce_precision(p1t, 8, 7) * -2.0
    km = lax.reduce_precision(p2t, 8, 7)

    qm2_sc = qm2[:, :, NTC:].reshape(B, 3, NBLK, RPB).transpose(0, 2, 1, 3)
    qraw_sc = p1t[:, :, NTC:].reshape(B, 3, NBLK, RPB).transpose(0, 2, 1, 3)
    d1m, d2p = _sc_part(qm2_sc, qraw_sc, km, p2t)
    s1, cmin = _tc_part(p1m2[:, :, :NTC], p1t[:, :, :NTC], p2t)
    return _merge(s1, cmin, d1m, d2p)
